# baseline (speedup 1.0000x reference)
import jax
import jax.numpy as jnp
from jax.experimental import pallas as pl
from jax.experimental.pallas import tpu as pltpu

_BB = 512


def _lstm_tile_kernel(x_ref, wih_ref, bias_ref, whh_ref, wout_ref, out_ref,
                      *, T: int, D: int):
    BB = x_ref.shape[0]
    H, H4 = whh_ref.shape

    bias = bias_ref[0:1, :]
    xg = [jnp.dot(x_ref[:, t * D:(t + 1) * D], wih_ref[...],
                  preferred_element_type=jnp.float32) + bias
          for t in range(T)]

    whh = whh_ref[...]
    h = jnp.zeros((BB, H), jnp.float32)
    c = jnp.zeros((BB, H), jnp.float32)
    for t in range(T):
        gates = xg[t] + jnp.dot(h, whh, preferred_element_type=jnp.float32)
        a = jnp.tanh(gates)
        i_g = a[:, 0 * H:1 * H] * 0.5 + 0.5
        f_g = a[:, 1 * H:2 * H] * 0.5 + 0.5
        g_g = a[:, 2 * H:3 * H]
        o_g = a[:, 3 * H:4 * H] * 0.5 + 0.5
        c = f_g * c + i_g * g_g
        h = o_g * jnp.tanh(c)

    out_ref[...] = (jnp.dot(h, wout_ref[0:H, :],
                            preferred_element_type=jnp.float32)
                    + wout_ref[H:H + 1, :]).astype(out_ref.dtype)


def kernel(x, w_ih, w_hh, b_ih, b_hh, w_out, b_out):
    B, T, D = x.shape
    H = w_hh.shape[0]
    A = w_out.shape[1]
    H4 = 4 * H

    BB = min(_BB, B)
    nb = -(-B // BB)
    Bp = nb * BB
    x2 = x.reshape(B, T * D)
    if Bp != B:
        x2 = jnp.pad(x2, ((0, Bp - B), (0, 0)))

    col = jnp.arange(H4)
    ifo = jnp.where((col >= 2 * H) & (col < 3 * H), 1.0, 0.5).astype(jnp.float32)
    wih_s = w_ih * ifo[None, :]
    whh_s = w_hh * ifo[None, :]
    bias_s = jnp.broadcast_to(((b_ih + b_hh) * ifo)[None, :], (8, H4))

    out_rows = ((H + 1 + 7) // 8) * 8
    wout = jnp.zeros((out_rows, A), jnp.float32)
    wout = wout.at[0:H, :].set(w_out)
    wout = wout.at[H, :].set(b_out)

    from functools import partial
    out_p = pl.pallas_call(
        partial(_lstm_tile_kernel, T=T, D=D),
        out_shape=jax.ShapeDtypeStruct((Bp, A), jnp.float32),
        grid=(nb,),
        in_specs=[
            pl.BlockSpec((BB, T * D), lambda i: (i, 0)),
            pl.BlockSpec((D, H4), lambda i: (0, 0)),
            pl.BlockSpec((8, H4), lambda i: (0, 0)),
            pl.BlockSpec((H, H4), lambda i: (0, 0)),
            pl.BlockSpec((out_rows, A), lambda i: (0, 0)),
        ],
        out_specs=pl.BlockSpec((BB, A), lambda i: (i, 0)),
        compiler_params=pltpu.CompilerParams(
            dimension_semantics=("parallel",)),
    )(x2, wih_s, bias_s, whh_s, wout)
    return out_p[:B]


# speedup vs baseline: 1.1388x; 1.1388x over previous
import jax
import jax.numpy as jnp
from jax.experimental import pallas as pl
from jax.experimental.pallas import tpu as pltpu

_BB = 2048


def _lstm_tile_kernel(x_ref, wih_ref, bias_ref, whh_ref, wout_ref, out_ref,
                      *, T: int, D: int):
    BB = x_ref.shape[0]
    H, H4 = whh_ref.shape

    bias = bias_ref[0:1, :]
    xg = [jnp.dot(x_ref[:, t * D:(t + 1) * D], wih_ref[...],
                  preferred_element_type=jnp.float32) + bias
          for t in range(T)]

    whh = whh_ref[...]
    h = jnp.zeros((BB, H), jnp.float32)
    c = jnp.zeros((BB, H), jnp.float32)
    for t in range(T):
        gates = xg[t] + jnp.dot(h, whh, preferred_element_type=jnp.float32)
        a = jnp.tanh(gates)
        i_g = a[:, 0 * H:1 * H] * 0.5 + 0.5
        f_g = a[:, 1 * H:2 * H] * 0.5 + 0.5
        g_g = a[:, 2 * H:3 * H]
        o_g = a[:, 3 * H:4 * H] * 0.5 + 0.5
        c = f_g * c + i_g * g_g
        h = o_g * jnp.tanh(c)

    out_ref[...] = (jnp.dot(h, wout_ref[0:H, :],
                            preferred_element_type=jnp.float32)
                    + wout_ref[H:H + 1, :]).astype(out_ref.dtype)


def kernel(x, w_ih, w_hh, b_ih, b_hh, w_out, b_out):
    B, T, D = x.shape
    H = w_hh.shape[0]
    A = w_out.shape[1]
    H4 = 4 * H

    BB = min(_BB, B)
    nb = -(-B // BB)
    Bp = nb * BB
    x2 = x.reshape(B, T * D)
    if Bp != B:
        x2 = jnp.pad(x2, ((0, Bp - B), (0, 0)))

    col = jnp.arange(H4)
    ifo = jnp.where((col >= 2 * H) & (col < 3 * H), 1.0, 0.5).astype(jnp.float32)
    wih_s = w_ih * ifo[None, :]
    whh_s = w_hh * ifo[None, :]
    bias_s = jnp.broadcast_to(((b_ih + b_hh) * ifo)[None, :], (8, H4))

    out_rows = ((H + 1 + 7) // 8) * 8
    wout = jnp.zeros((out_rows, A), jnp.float32)
    wout = wout.at[0:H, :].set(w_out)
    wout = wout.at[H, :].set(b_out)

    from functools import partial
    out_p = pl.pallas_call(
        partial(_lstm_tile_kernel, T=T, D=D),
        out_shape=jax.ShapeDtypeStruct((Bp, A), jnp.float32),
        grid=(nb,),
        in_specs=[
            pl.BlockSpec((BB, T * D), lambda i: (i, 0)),
            pl.BlockSpec((D, H4), lambda i: (0, 0)),
            pl.BlockSpec((8, H4), lambda i: (0, 0)),
            pl.BlockSpec((H, H4), lambda i: (0, 0)),
            pl.BlockSpec((out_rows, A), lambda i: (0, 0)),
        ],
        out_specs=pl.BlockSpec((BB, A), lambda i: (i, 0)),
        compiler_params=pltpu.CompilerParams(
            dimension_semantics=("parallel",)),
    )(x2, wih_s, bias_s, whh_s, wout)
    return out_p[:B]


# speedup vs baseline: 1.9719x; 1.7316x over previous
from functools import partial

import jax
import jax.numpy as jnp
from jax import lax
from jax.experimental import pallas as pl
from jax.experimental.pallas import tpu as pltpu

_BB = 2048


def _lstm_tile_kernel(x_ref, wih_ref, whh_ref, wout_ref, out_ref, *, T: int):
    BB = x_ref.shape[0]
    D = x_ref.shape[2]
    H4 = wih_ref.shape[1]
    H = H4 // 4

    lane = lax.broadcasted_iota(jnp.int32, (1, H4), 1)
    is_g = (lane >= 2 * H) & (lane < 3 * H)
    cs = jnp.where(is_g, 1.0, 0.5)

    wih = wih_ref[...] * cs
    whh = whh_ref[0:H, :] * cs
    bias = (whh_ref[H:H + 1, :] + whh_ref[H + 1:H + 2, :]) * cs

    xg = [jnp.dot(x_ref[:, t, :], wih,
                  preferred_element_type=jnp.float32) + bias
          for t in range(T)]

    h = jnp.zeros((BB, H), jnp.float32)
    c = jnp.zeros((BB, H), jnp.float32)
    for t in range(T):
        gates = xg[t] + jnp.dot(h, whh, preferred_element_type=jnp.float32)
        a = jnp.tanh(gates)
        i_g = a[:, 0 * H:1 * H] * 0.5 + 0.5
        f_g = a[:, 1 * H:2 * H] * 0.5 + 0.5
        g_g = a[:, 2 * H:3 * H]
        o_g = a[:, 3 * H:4 * H] * 0.5 + 0.5
        c = f_g * c + i_g * g_g
        h = o_g * jnp.tanh(c)

    out_ref[...] = (jnp.dot(h, wout_ref[0:H, :],
                            preferred_element_type=jnp.float32)
                    + wout_ref[H:H + 1, :]).astype(out_ref.dtype)


def kernel(x, w_ih, w_hh, b_ih, b_hh, w_out, b_out):
    B, T, D = x.shape
    H = w_hh.shape[0]
    A = w_out.shape[1]
    H4 = 4 * H

    BB = min(_BB, B)
    nb = -(-B // BB)
    Bp = nb * BB
    if Bp != B:
        x = jnp.pad(x, ((0, Bp - B), (0, 0), (0, 0)))

    hh_rows = ((H + 2 + 7) // 8) * 8
    whh_slab = jnp.zeros((hh_rows, H4), jnp.float32)
    whh_slab = whh_slab.at[0:H, :].set(w_hh)
    whh_slab = whh_slab.at[H, :].set(b_ih)
    whh_slab = whh_slab.at[H + 1, :].set(b_hh)

    out_rows = ((H + 1 + 7) // 8) * 8
    wout_slab = jnp.zeros((out_rows, A), jnp.float32)
    wout_slab = wout_slab.at[0:H, :].set(w_out)
    wout_slab = wout_slab.at[H, :].set(b_out)

    out_p = pl.pallas_call(
        partial(_lstm_tile_kernel, T=T),
        out_shape=jax.ShapeDtypeStruct((Bp, A), jnp.float32),
        grid=(nb,),
        in_specs=[
            pl.BlockSpec((BB, T, D), lambda i: (i, 0, 0)),
            pl.BlockSpec((D, H4), lambda i: (0, 0)),
            pl.BlockSpec((hh_rows, H4), lambda i: (0, 0)),
            pl.BlockSpec((out_rows, A), lambda i: (0, 0)),
        ],
        out_specs=pl.BlockSpec((BB, A), lambda i: (i, 0)),
        compiler_params=pltpu.CompilerParams(
            dimension_semantics=("parallel",)),
    )(x, w_ih, whh_slab, wout_slab)
    return out_p[:B]


# speedup vs baseline: 2.0513x; 1.0402x over previous
from functools import partial

import jax
import jax.numpy as jnp
from jax import lax
from jax.experimental import pallas as pl
from jax.experimental.pallas import tpu as pltpu

_BB = 1024


def _lstm_tile_kernel(x_ref, wih_ref, whh_ref, wout_ref, out_ref, *, T: int):
    BB = x_ref.shape[0]
    D = x_ref.shape[2]
    H4 = wih_ref.shape[1]
    H = H4 // 4

    lane = lax.broadcasted_iota(jnp.int32, (1, H4), 1)
    is_g = (lane >= 2 * H) & (lane < 3 * H)
    cs = jnp.where(is_g, 1.0, 0.5)

    wih = wih_ref[...] * cs
    whh = whh_ref[0:H, :] * cs
    bias = (whh_ref[H:H + 1, :] + whh_ref[H + 1:H + 2, :]) * cs

    xg = [jnp.dot(x_ref[:, t, :], wih,
                  preferred_element_type=jnp.float32) + bias
          for t in range(T)]

    h = jnp.zeros((BB, H), jnp.float32)
    c = jnp.zeros((BB, H), jnp.float32)
    for t in range(T):
        gates = xg[t] + jnp.dot(h, whh, preferred_element_type=jnp.float32)
        a = jnp.tanh(gates)
        i_g = a[:, 0 * H:1 * H] * 0.5 + 0.5
        f_g = a[:, 1 * H:2 * H] * 0.5 + 0.5
        g_g = a[:, 2 * H:3 * H]
        o_g = a[:, 3 * H:4 * H] * 0.5 + 0.5
        c = f_g * c + i_g * g_g
        h = o_g * jnp.tanh(c)

    out_ref[...] = (jnp.dot(h, wout_ref[0:H, :],
                            preferred_element_type=jnp.float32)
                    + wout_ref[H:H + 1, :]).astype(out_ref.dtype)


def kernel(x, w_ih, w_hh, b_ih, b_hh, w_out, b_out):
    B, T, D = x.shape
    H = w_hh.shape[0]
    A = w_out.shape[1]
    H4 = 4 * H

    BB = min(_BB, B)
    nb = -(-B // BB)
    Bp = nb * BB
    if Bp != B:
        x = jnp.pad(x, ((0, Bp - B), (0, 0), (0, 0)))

    hh_rows = ((H + 2 + 7) // 8) * 8
    whh_slab = jnp.zeros((hh_rows, H4), jnp.float32)
    whh_slab = whh_slab.at[0:H, :].set(w_hh)
    whh_slab = whh_slab.at[H, :].set(b_ih)
    whh_slab = whh_slab.at[H + 1, :].set(b_hh)

    out_rows = ((H + 1 + 7) // 8) * 8
    wout_slab = jnp.zeros((out_rows, A), jnp.float32)
    wout_slab = wout_slab.at[0:H, :].set(w_out)
    wout_slab = wout_slab.at[H, :].set(b_out)

    out_p = pl.pallas_call(
        partial(_lstm_tile_kernel, T=T),
        out_shape=jax.ShapeDtypeStruct((Bp, A), jnp.float32),
        grid=(nb,),
        in_specs=[
            pl.BlockSpec((BB, T, D), lambda i: (i, 0, 0)),
            pl.BlockSpec((D, H4), lambda i: (0, 0)),
            pl.BlockSpec((hh_rows, H4), lambda i: (0, 0)),
            pl.BlockSpec((out_rows, A), lambda i: (0, 0)),
        ],
        out_specs=pl.BlockSpec((BB, A), lambda i: (i, 0)),
        compiler_params=pltpu.CompilerParams(
            dimension_semantics=("parallel",)),
    )(x, w_ih, whh_slab, wout_slab)
    return out_p[:B]
